# baseline (speedup 1.0000x reference)
import jax
import jax.numpy as jnp
from jax.experimental import pallas as pl

L = 2
E = 8
K = 2


def _identity_kernel(x_ref, o_ref):
    o_ref[...] = x_ref[...]


def _gru(x, Wih, Whh, bih, bhh):
    Bx, Sx, Hx = x.shape

    def step(h, xt):
        gi = xt @ Wih.T + bih
        gh = h @ Whh.T + bhh
        i_r, i_z, i_n = jnp.split(gi, 3, axis=-1)
        h_r, h_z, h_n = jnp.split(gh, 3, axis=-1)
        r = jax.nn.sigmoid(i_r + h_r)
        z = jax.nn.sigmoid(i_z + h_z)
        n = jnp.tanh(i_n + r * h_n)
        h_new = (1.0 - z) * n + z * h
        return h_new, h_new

    h0 = jnp.zeros((Bx, Hx), x.dtype)
    xs = jnp.swapaxes(x, 0, 1)
    _, ys = jax.lax.scan(step, h0, xs)
    return jnp.swapaxes(ys, 0, 1)


def kernel(x, proj_W, proj_b, gru_Wih, gru_Whh, gru_bih, gru_bhh,
           router_W, router_b, exp_W1, exp_b1, exp_W2, exp_b2):
    out = x
    for l in range(L):
        h = out @ proj_W[l] + proj_b[l]
        h = _gru(h, gru_Wih, gru_Whh, gru_bih, gru_bhh)
        logits = h @ router_W[l] + router_b[l]
        Bx, Sx, Dx = out.shape
        T = Bx * Sx
        xf = out.reshape(T, Dx)
        lf = logits.reshape(T, E)
        topv, topi = jax.lax.top_k(lf, K)
        gates = jax.nn.softmax(topv, axis=-1)
        gate_full = jnp.zeros((T, E), jnp.float32).at[jnp.arange(T)[:, None], topi].add(gates)
        hmid = jax.nn.relu(jnp.einsum('td,edf->tef', xf, exp_W1[l]) + exp_b1[l])
        eo = jnp.einsum('tef,efd->ted', hmid, exp_W2[l]) + exp_b2[l]
        comb = jnp.einsum('te,ted->td', gate_full, eo)
        out = out + comb.reshape(Bx, Sx, Dx)
    out = pl.pallas_call(
        _identity_kernel,
        out_shape=jax.ShapeDtypeStruct(out.shape, out.dtype),
    )(out)
    return out


# speedup vs baseline: 3.9431x; 3.9431x over previous
import jax
import jax.numpy as jnp
from jax.experimental import pallas as pl
from jax.experimental.pallas import tpu as pltpu

L = 2
E = 8
K = 2
D = 768
H = 768
S = 2048
F = 4 * D

NEG = -1e30
HIGH = jax.lax.Precision.HIGHEST


def _b16(x):
    return x.astype(jnp.bfloat16).astype(jnp.float32)


def _h0_body(x_ref, pW_ref, pb_ref, h0_ref):
    h0_ref[...] = jnp.dot(_b16(x_ref[...]), _b16(pW_ref[...]),
                          preferred_element_type=jnp.float32,
                          precision=HIGH) + pb_ref[...]


def _h0_matmul(x2d, pW, pb):
    TB = 512
    return pl.pallas_call(
        _h0_body,
        grid=(S // TB,),
        in_specs=[
            pl.BlockSpec((TB, D), lambda t: (t, 0)),
            pl.BlockSpec((D, H), lambda t: (0, 0)),
            pl.BlockSpec((1, H), lambda t: (0, 0)),
        ],
        out_specs=pl.BlockSpec((TB, H), lambda t: (t, 0)),
        out_shape=jax.ShapeDtypeStruct((S, H), jnp.float32),
    )(x2d, pW, pb.reshape(1, H))


def _gru_body(h0_ref, Wih_ref, bih_ref, Whh_ref, bhh_ref, h_ref):
    def step8(t8, h):
        base = pl.multiple_of(t8 * 8, 8)
        h0_tile = h0_ref[pl.ds(base, 8), :]
        rows = []
        for i in range(8):
            gi_t = jnp.dot(h0_tile[i:i + 1, :], Wih_ref[...],
                           preferred_element_type=jnp.float32)
            gi_t = gi_t + bih_ref[...]
            gh = jnp.dot(h, Whh_ref[...], preferred_element_type=jnp.float32)
            gh = gh + bhh_ref[...]
            r = 1.0 / (1.0 + jnp.exp(-(gi_t[:, :H] + gh[:, :H])))
            z = 1.0 / (1.0 + jnp.exp(-(gi_t[:, H:2 * H] + gh[:, H:2 * H])))
            n = jnp.tanh(gi_t[:, 2 * H:] + r * gh[:, 2 * H:])
            h = (1.0 - z) * n + z * h
            rows.append(h)
        h_ref[pl.ds(base, 8), :] = jnp.concatenate(rows, axis=0)
        return h

    jax.lax.fori_loop(0, S // 8, step8, jnp.zeros((1, H), jnp.float32))


def _gru_h(x2d, pW, pb, WihT, bih, WhhT, bhh):
    h0 = _h0_matmul(x2d, pW, pb)
    Wih_b = WihT.astype(jnp.bfloat16).astype(jnp.float32)
    Whh_b = WhhT.astype(jnp.bfloat16).astype(jnp.float32)
    return pl.pallas_call(
        _gru_body,
        out_shape=jax.ShapeDtypeStruct((S, H), jnp.float32),
    )(h0, Wih_b, bih.reshape(1, 3 * H), Whh_b, bhh.reshape(1, 3 * H))


def _logits_body(h_ref, rW_ref, rb_ref, logits_ref):
    logits_ref[...] = jnp.dot(_b16(h_ref[...]), _b16(rW_ref[...]),
                              preferred_element_type=jnp.float32,
                              precision=HIGH) + rb_ref[...]


def _logits(h, rW, rb):
    return pl.pallas_call(
        _logits_body,
        out_shape=jax.ShapeDtypeStruct((S, E), jnp.float32),
    )(h, rW, rb.reshape(1, E))


def _top2_gate(lg, e):
    cols = jax.lax.broadcasted_iota(jnp.int32, lg.shape, 1)
    v1 = jnp.max(lg, axis=1, keepdims=True)
    i1 = jnp.min(jnp.where(lg == v1, cols, E), axis=1, keepdims=True)
    masked = jnp.where(cols == i1, NEG, lg)
    v2 = jnp.max(masked, axis=1, keepdims=True)
    i2 = jnp.min(jnp.where(masked == v2, cols, E), axis=1, keepdims=True)
    ex = jnp.exp(v2 - v1)
    s = 1.0 + ex
    g1 = 1.0 / s
    g2 = ex / s
    return jnp.where(i1 == e, g1, 0.0) + jnp.where(i2 == e, g2, 0.0)


def _dense_expert_body(x_ref, logits_ref, W1_ref, b1_ref, W2_ref, b2_ref,
                       out_ref):
    e = pl.program_id(1)
    gate = _top2_gate(logits_ref[...], e)

    x = _b16(x_ref[...])
    acc = jnp.zeros((x.shape[0], D), jnp.float32)
    for c in range(4):
        hm = jnp.maximum(
            jnp.dot(x, _b16(W1_ref[0, :, c * D:(c + 1) * D]),
                    preferred_element_type=jnp.float32, precision=HIGH)
            + b1_ref[0, :, c * D:(c + 1) * D], 0.0)
        acc = acc + jnp.dot(_b16(hm), _b16(W2_ref[0, c * D:(c + 1) * D, :]),
                            preferred_element_type=jnp.float32, precision=HIGH)
    contrib = _b16(acc + b2_ref[0]) * _b16(gate)

    @pl.when(e == 0)
    def _init():
        out_ref[...] = x_ref[...] + contrib

    @pl.when(e != 0)
    def _acc():
        out_ref[...] = out_ref[...] + contrib


def _dense_experts(x2d, logits, W1, b1, W2, b2):
    TB = 512
    nb = S // TB
    return pl.pallas_call(
        _dense_expert_body,
        grid=(nb, E),
        in_specs=[
            pl.BlockSpec((TB, D), lambda t, e: (t, 0)),
            pl.BlockSpec((TB, E), lambda t, e: (t, 0)),
            pl.BlockSpec((1, D, F), lambda t, e: (e, 0, 0)),
            pl.BlockSpec((1, 1, F), lambda t, e: (e, 0, 0)),
            pl.BlockSpec((1, F, D), lambda t, e: (e, 0, 0)),
            pl.BlockSpec((1, 1, D), lambda t, e: (e, 0, 0)),
        ],
        out_specs=pl.BlockSpec((TB, D), lambda t, e: (t, 0)),
        out_shape=jax.ShapeDtypeStruct((S, D), jnp.float32),
    )(x2d, logits, W1, b1.reshape(E, 1, F), W2, b2.reshape(E, 1, D))


def kernel(x, proj_W, proj_b, gru_Wih, gru_Whh, gru_bih, gru_bhh,
           router_W, router_b, exp_W1, exp_b1, exp_W2, exp_b2):
    WihT = gru_Wih.T
    WhhT = gru_Whh.T
    out = x.reshape(S, D)
    for l in range(L):
        h = _gru_h(out, proj_W[l], proj_b[l], WihT, gru_bih, WhhT, gru_bhh)
        logits = _logits(h, router_W[l], router_b[l])
        out = _dense_experts(out, logits, exp_W1[l], exp_b1[l],
                             exp_W2[l], exp_b2[l])
    return out.reshape(x.shape)


# speedup vs baseline: 5.0443x; 1.2793x over previous
import functools

import jax
import jax.numpy as jnp
from jax import lax
from jax.experimental import pallas as pl
from jax.experimental.pallas import tpu as pltpu
from jax.experimental.pallas import tpu_sc as plsc

L = 2
E = 8
K = 2
D = 768
H = 768
S = 2048
F = 4 * D

TKB = 128
NPB = S * K // TKB + E
P = NPB * TKB

NW = 32
PPW = S * K // NW

NEG = -1e30
HIGH = jax.lax.Precision.HIGHEST


def _b16(x):
    return x.astype(jnp.bfloat16).astype(jnp.float32)


def _h0_body(x_ref, pW_ref, pb_ref, h0_ref):
    h0_ref[...] = jnp.dot(_b16(x_ref[...]), _b16(pW_ref[...]),
                          preferred_element_type=jnp.float32,
                          precision=HIGH) + pb_ref[...]


def _h0_matmul(x2d, pW, pb):
    TB = 512
    return pl.pallas_call(
        _h0_body,
        grid=(S // TB,),
        in_specs=[
            pl.BlockSpec((TB, D), lambda t: (t, 0)),
            pl.BlockSpec((D, H), lambda t: (0, 0)),
            pl.BlockSpec((1, H), lambda t: (0, 0)),
        ],
        out_specs=pl.BlockSpec((TB, H), lambda t: (t, 0)),
        out_shape=jax.ShapeDtypeStruct((S, H), jnp.float32),
    )(x2d, pW, pb.reshape(1, H))


def _tanh_xla(x):
    tiny = jnp.abs(x) < 0.0004
    xc = jnp.clip(x, -7.90531110763549805, 7.90531110763549805)
    x2 = xc * xc
    p = x2 * (-2.76076847742355e-16) + 2.00018790482477e-13
    p = x2 * p + (-8.60467152213735e-11)
    p = x2 * p + 5.12229709037114e-07
    p = x2 * p + 1.48572235717979e-05
    p = x2 * p + 6.37261928875436e-04
    p = x2 * p + 4.89352455891786e-03
    p = xc * p
    q = x2 * 1.19825839466702e-06 + 1.18534705686654e-04
    q = x2 * q + 2.26843463243900e-03
    q = x2 * q + 4.89352518554385e-03
    return jnp.where(tiny, x, p / q)


def _sigmoid_xla(x):
    return 0.5 * _tanh_xla(0.5 * x) + 0.5

def _gru_body(h0_ref, Wih_ref, bih_ref, Whh_ref, bhh_ref, h_ref):
    def step8(t8, h):
        base = pl.multiple_of(t8 * 8, 8)
        h0_tile = h0_ref[pl.ds(base, 8), :]
        rows = []
        for i in range(8):
            gi_t = jnp.dot(h0_tile[i:i + 1, :], Wih_ref[...],
                           preferred_element_type=jnp.float32)
            gi_t = gi_t + bih_ref[...]
            gh = jnp.dot(h, Whh_ref[...], preferred_element_type=jnp.float32)
            gh = gh + bhh_ref[...]
            r = 1.0 / (1.0 + jnp.exp(-(gi_t[:, :H] + gh[:, :H])))
            z = 1.0 / (1.0 + jnp.exp(-(gi_t[:, H:2 * H] + gh[:, H:2 * H])))
            n = jnp.tanh(gi_t[:, 2 * H:] + r * gh[:, 2 * H:])
            h = (1.0 - z) * n + z * h
            rows.append(h)
        h_ref[pl.ds(base, 8), :] = jnp.concatenate(rows, axis=0)
        return h

    jax.lax.fori_loop(0, S // 8, step8, jnp.zeros((1, H), jnp.float32))


def _gru_h(x2d, pW, pb, WihT, bih, WhhT, bhh):
    h0 = _h0_matmul(x2d, pW, pb)
    Wih_b = WihT.astype(jnp.bfloat16).astype(jnp.float32)
    Whh_b = WhhT.astype(jnp.bfloat16).astype(jnp.float32)
    return pl.pallas_call(
        _gru_body,
        out_shape=jax.ShapeDtypeStruct((S, H), jnp.float32),
    )(h0, Wih_b, bih.reshape(1, 3 * H), Whh_b, bhh.reshape(1, 3 * H))


def _routing_body(h_ref, rW_ref, rb_ref, gates_ref, pos_ref, be_ref):
    lg = jnp.dot(_b16(h_ref[...]), _b16(rW_ref[...]),
                 preferred_element_type=jnp.float32, precision=HIGH) + rb_ref[...]
    cols = jax.lax.broadcasted_iota(jnp.int32, lg.shape, 1)
    v1 = jnp.max(lg, axis=1, keepdims=True)
    i1 = jnp.min(jnp.where(lg == v1, cols, E), axis=1, keepdims=True)
    masked = jnp.where(cols == i1, NEG, lg)
    v2 = jnp.max(masked, axis=1, keepdims=True)
    i2 = jnp.min(jnp.where(masked == v2, cols, E), axis=1, keepdims=True)
    ex = jnp.exp(v2 - v1)
    sden = 1.0 + ex
    g1 = 1.0 / sden
    g2 = ex / sden
    ones_l = jnp.ones((1, 128), jnp.float32)
    gates_ref[...] = jnp.concatenate(
        [(g1 * ones_l).reshape(1, S, 128), (g2 * ones_l).reshape(1, S, 128)],
        axis=0)

    oh = jnp.concatenate(
        [(cols == i1).astype(jnp.float32), (cols == i2).astype(jnp.float32)],
        axis=0)

    CB = 512
    rows = jax.lax.broadcasted_iota(jnp.int32, (CB, CB), 0)
    colsq = jax.lax.broadcasted_iota(jnp.int32, (CB, CB), 1)
    tril = (rows > colsq).astype(jnp.float32)
    base = jnp.zeros((1, E), jnp.float32)
    ranks = []
    for b in range(2 * S // CB):
        blk = oh[b * CB:(b + 1) * CB]
        ranks.append(jnp.dot(tril, blk, preferred_element_type=jnp.float32,
                             precision=HIGH) + base)
        base = base + jnp.sum(blk, axis=0, keepdims=True)
    rank = jnp.concatenate(ranks, axis=0)
    counts = base

    padded = jnp.ceil(counts / TKB) * TKB
    eup = jax.lax.broadcasted_iota(jnp.int32, (E, E), 0)
    evp = jax.lax.broadcasted_iota(jnp.int32, (E, E), 1)
    upper = (eup < evp).astype(jnp.float32)
    poffs = jnp.dot(padded, upper, preferred_element_type=jnp.float32,
                    precision=HIGH)

    pos = jnp.sum(oh * (rank + poffs), axis=1, keepdims=True)
    pos_ref[...] = pos.astype(jnp.int32)

    jrow = jax.lax.broadcasted_iota(jnp.int32, (NPB, E), 0) * TKB
    be = jnp.sum((poffs.astype(jnp.int32) <= jrow).astype(jnp.int32),
                 axis=1, keepdims=True) - 1
    be_ref[...] = be


def _routing(h, rW, rb):
    return pl.pallas_call(
        _routing_body,
        out_shape=(
            jax.ShapeDtypeStruct((K, S, 128), jnp.float32),
            jax.ShapeDtypeStruct((K * S, 1), jnp.int32),
            jax.ShapeDtypeStruct((NPB, 1), jnp.int32),
        ),
    )(h, rW, rb.reshape(1, E))


def _dispatch_sc_body(x_hbm, pos_hbm, g_hbm, xs_hbm, gs_hbm,
                      idx_v, rows_v, g_v, sem):
    wid = lax.axis_index("s") * 2 + lax.axis_index("c")
    base = wid * PPW
    k = base // S
    tok0 = base - k * S
    pltpu.sync_copy(pos_hbm.at[pl.ds(base, PPW)], idx_v)
    pltpu.sync_copy(x_hbm.at[pl.ds(tok0, PPW)], rows_v)
    pltpu.sync_copy(g_hbm.at[k, pl.ds(tok0, PPW)], g_v)
    pltpu.async_copy(rows_v, xs_hbm.at[idx_v], sem).wait()
    pltpu.async_copy(g_v, gs_hbm.at[idx_v], sem).wait()


def _dispatch_sc(x2d, pos, gates):
    mesh = plsc.VectorSubcoreMesh(core_axis_name="c", subcore_axis_name="s")
    fn = functools.partial(
        pl.kernel, mesh=mesh,
        out_type=(
            jax.ShapeDtypeStruct((P, D), jnp.float32),
            jax.ShapeDtypeStruct((P, 128), jnp.float32),
        ),
        scratch_types=[
            pltpu.VMEM((PPW,), jnp.int32),
            pltpu.VMEM((PPW, D), jnp.float32),
            pltpu.VMEM((PPW, 128), jnp.float32),
            pltpu.SemaphoreType.DMA,
        ],
    )(_dispatch_sc_body)
    return fn(x2d, pos, gates)


def _expert_body(be_ref, xs_ref, gs_ref, W1_ref, b1_ref, W2_ref, b2_ref,
                 eo_ref):
    x = _b16(xs_ref[...])
    acc = jnp.zeros((TKB, D), jnp.float32)
    for c in range(4):
        hm = jnp.maximum(
            jnp.dot(x, _b16(W1_ref[0, :, c * D:(c + 1) * D]),
                    preferred_element_type=jnp.float32, precision=HIGH)
            + b1_ref[0, :, c * D:(c + 1) * D], 0.0)
        acc = acc + jnp.dot(_b16(hm), _b16(W2_ref[0, c * D:(c + 1) * D, :]),
                            preferred_element_type=jnp.float32, precision=HIGH)
    eo_ref[...] = _b16(acc + b2_ref[0]) * _b16(gs_ref[:, :1])


def _experts(xs, gs, W1, b1, W2, b2, be):
    grid_spec = pltpu.PrefetchScalarGridSpec(
        num_scalar_prefetch=1,
        grid=(NPB,),
        in_specs=[
            pl.BlockSpec((TKB, D), lambda j, be: (j, 0)),
            pl.BlockSpec((TKB, 128), lambda j, be: (j, 0)),
            pl.BlockSpec((1, D, F), lambda j, be: (be[j], 0, 0)),
            pl.BlockSpec((1, 1, F), lambda j, be: (be[j], 0, 0)),
            pl.BlockSpec((1, F, D), lambda j, be: (be[j], 0, 0)),
            pl.BlockSpec((1, 1, D), lambda j, be: (be[j], 0, 0)),
        ],
        out_specs=pl.BlockSpec((TKB, D), lambda j, be: (j, 0)),
    )
    return pl.pallas_call(
        _expert_body,
        grid_spec=grid_spec,
        out_shape=jax.ShapeDtypeStruct((P, D), jnp.float32),
    )(be, xs, gs, W1, b1.reshape(E, 1, F), W2, b2.reshape(E, 1, D))


def _combine_sc_body(eo_hbm, pos_hbm, r_hbm, idx_v, rows_v, sem):
    wid = lax.axis_index("s") * 2 + lax.axis_index("c")
    base = wid * PPW
    for c in range(2):
        off = base + c * (PPW // 2)
        pltpu.sync_copy(pos_hbm.at[pl.ds(off, PPW // 2)], idx_v)
        pltpu.async_copy(eo_hbm.at[idx_v], rows_v, sem).wait()
        pltpu.sync_copy(rows_v, r_hbm.at[pl.ds(off, PPW // 2)])


def _combine_sc(eo, pos):
    mesh = plsc.VectorSubcoreMesh(core_axis_name="c", subcore_axis_name="s")
    fn = functools.partial(
        pl.kernel, mesh=mesh,
        out_type=jax.ShapeDtypeStruct((K * S, D), jnp.float32),
        scratch_types=[
            pltpu.VMEM((PPW // 2,), jnp.int32),
            pltpu.VMEM((PPW // 2, D), jnp.float32),
            pltpu.SemaphoreType.DMA,
        ],
    )(_combine_sc_body)
    return fn(eo, pos)


def _add_body(x_ref, r0_ref, r1_ref, out_ref):
    out_ref[...] = x_ref[...] + r0_ref[...] + r1_ref[...]


def _residual_add(x2d, r):
    TB = 512
    return pl.pallas_call(
        _add_body,
        grid=(S // TB,),
        in_specs=[
            pl.BlockSpec((TB, D), lambda t: (t, 0)),
            pl.BlockSpec((TB, D), lambda t: (t, 0)),
            pl.BlockSpec((TB, D), lambda t: (t + S // TB, 0)),
        ],
        out_specs=pl.BlockSpec((TB, D), lambda t: (t, 0)),
        out_shape=jax.ShapeDtypeStruct((S, D), jnp.float32),
    )(x2d, r, r)


def _moe_layer(out, h, rW, rb, W1, b1, W2, b2):
    gates, pos, be = _routing(h, rW, rb)
    pos_flat = pos.reshape(K * S)
    xs, gs = _dispatch_sc(out, pos_flat, gates)
    eo = _experts(xs, gs, W1, b1, W2, b2, be.reshape(NPB))
    r = _combine_sc(eo, pos_flat)
    return _residual_add(out, r)


def kernel(x, proj_W, proj_b, gru_Wih, gru_Whh, gru_bih, gru_bhh,
           router_W, router_b, exp_W1, exp_b1, exp_W2, exp_b2):
    WihT = gru_Wih.T
    WhhT = gru_Whh.T
    out = x.reshape(S, D)
    for l in range(L):
        h = _gru_h(out, proj_W[l], proj_b[l], WihT, gru_bih, WhhT, gru_bhh)
        out = _moe_layer(out, h, router_W[l], router_b[l], exp_W1[l],
                         exp_b1[l], exp_W2[l], exp_b2[l])
    return out.reshape(x.shape)
